# Initial kernel scaffold; baseline (speedup 1.0000x reference)
import jax
import jax.numpy as jnp
from jax.experimental import pallas as pl


def kernel(x, table):
    raise NotImplementedError("write your pallas kernel here")


# speedup vs baseline: 4.9053x; 4.9053x over previous
import functools

import jax
import jax.numpy as jnp
from jax import lax
from jax.experimental import pallas as pl
from jax.experimental.pallas import tpu as pltpu
from jax.experimental.pallas import tpu_sc as plsc

_BATCH = 16384
_HIST = 200
_EMBED = 32
_N = _BATCH * _HIST

_NC = 2
_NS = 16
_NW = _NC * _NS
_BPW = _N // _NW
_C = 1600
_NCHUNK = _BPW // _C

_mesh = plsc.VectorSubcoreMesh(core_axis_name="c", subcore_axis_name="s")


@functools.partial(
    pl.kernel,
    out_type=jax.ShapeDtypeStruct((_N, _EMBED), jnp.float32),
    mesh=_mesh,
    scratch_types=[
        pltpu.VMEM((_C,), jnp.int32),
        pltpu.VMEM((_C, _EMBED), jnp.float32),
        pltpu.SemaphoreType.DMA,
    ],
    compiler_params=pltpu.CompilerParams(use_tc_tiling_on_sc=False),
)
def _gather_kernel(idx_hbm, table_hbm, out_hbm, idx_v, rows_v, sem):
    wid = lax.axis_index("s") * _NC + lax.axis_index("c")
    base = wid * _BPW

    def chunk(j, carry):
        off = base + j * _C
        pltpu.sync_copy(idx_hbm.at[pl.ds(off, _C)], idx_v)
        pltpu.async_copy(table_hbm.at[idx_v], rows_v, sem).wait()
        pltpu.sync_copy(rows_v, out_hbm.at[pl.ds(off, _C)])
        return carry

    lax.fori_loop(0, _NCHUNK, chunk, 0)


def kernel(x, table):
    flat = x.reshape(-1).astype(jnp.int32)
    out = _gather_kernel(flat, table)
    return out.reshape(_BATCH, _HIST, _EMBED)


# speedup vs baseline: 5.0373x; 1.0269x over previous
import functools

import jax
import jax.numpy as jnp
from jax import lax
from jax.experimental import pallas as pl
from jax.experimental.pallas import tpu as pltpu
from jax.experimental.pallas import tpu_sc as plsc

_BATCH = 16384
_HIST = 200
_EMBED = 32
_N = _BATCH * _HIST

_NC = 2
_NS = 16
_NW = _NC * _NS
_BPW = _N // _NW
_C = 1600
_NCHUNK = _BPW // _C

_mesh = plsc.VectorSubcoreMesh(core_axis_name="c", subcore_axis_name="s")


@functools.partial(
    pl.kernel,
    out_type=jax.ShapeDtypeStruct((_N, _EMBED), jnp.float32),
    mesh=_mesh,
    scratch_types=[
        pltpu.VMEM((2, _C), jnp.int32),
        pltpu.VMEM((2, _C, _EMBED), jnp.float32),
        pltpu.SemaphoreType.DMA,
        pltpu.SemaphoreType.DMA,
        pltpu.SemaphoreType.DMA,
        pltpu.SemaphoreType.DMA,
        pltpu.SemaphoreType.DMA,
        pltpu.SemaphoreType.DMA,
    ],
    compiler_params=pltpu.CompilerParams(use_tc_tiling_on_sc=False),
)
def _gather_kernel(idx_hbm, table_hbm, out_hbm, idx_v, rows_v,
                   sem_l0, sem_l1, sem_g0, sem_g1, sem_w0, sem_w1):
    wid = lax.axis_index("s") * _NC + lax.axis_index("c")
    base = wid * _BPW
    sem_l = (sem_l0, sem_l1)
    sem_g = (sem_g0, sem_g1)
    sem_w = (sem_w0, sem_w1)

    def l_copy(j, b):
        return pltpu.make_async_copy(
            idx_hbm.at[pl.ds(base + j * _C, _C)], idx_v.at[b], sem_l[b])

    def g_copy(b):
        return pltpu.make_async_copy(
            table_hbm.at[idx_v.at[b]], rows_v.at[b], sem_g[b])

    def w_copy(j, b):
        return pltpu.make_async_copy(
            rows_v.at[b], out_hbm.at[pl.ds(base + j * _C, _C)], sem_w[b])

    l_copy(0, 0).start()
    l_copy(1, 1).start()
    l_copy(0, 0).wait()
    g_copy(0).start()
    g_copy(0).wait()
    l_copy(1, 1).wait()
    g_copy(1).start()
    w_copy(0, 0).start()
    l_copy(2, 0).start()
    g_copy(1).wait()
    w_copy(0, 0).wait()
    l_copy(2, 0).wait()
    g_copy(0).start()
    w_copy(1, 1).start()
    l_copy(3, 1).start()

    def body(jj, carry):
        j = 2 * jj
        g_copy(0).wait()
        w_copy(j - 1, 1).wait()
        l_copy(j + 1, 1).wait()
        g_copy(1).start()
        w_copy(j, 0).start()
        l_copy(j + 2, 0).start()
        g_copy(1).wait()
        w_copy(j, 0).wait()
        l_copy(j + 2, 0).wait()
        g_copy(0).start()
        w_copy(j + 1, 1).start()
        l_copy(j + 3, 1).start()
        return carry

    lax.fori_loop(1, _NCHUNK // 2 - 1, body, 0)

    jl = _NCHUNK - 2
    g_copy(0).wait()
    w_copy(jl - 1, 1).wait()
    l_copy(jl + 1, 1).wait()
    g_copy(1).start()
    w_copy(jl, 0).start()
    g_copy(1).wait()
    w_copy(jl, 0).wait()
    w_copy(jl + 1, 1).start()
    w_copy(jl + 1, 1).wait()


def kernel(x, table):
    flat = x.reshape(-1).astype(jnp.int32)
    out = _gather_kernel(flat, table)
    return out.reshape(_BATCH, _HIST, _EMBED)
